# baseline (speedup 1.0000x reference)
import functools

import jax
import jax.numpy as jnp
from jax import lax
from jax.experimental import pallas as pl
from jax.experimental.pallas import tpu as pltpu
from jax.experimental.pallas import tpu_sc as plsc

N_NODES = 10000
D = 128
N_EDGES = 320000
NC = 2
NS = 16
NW = NC * NS
CH = 128
RPT = 80
RPT0 = 160
RPT1 = 0
TOT_ROWS = NS * (RPT0 + RPT1)
KG = 8
E_PAD = TOT_ROWS * CH
N_ACC = 10112
PER = N_ACC // NS
CW = 128


def _make_seg():
    mesh = plsc.VectorSubcoreMesh(core_axis_name="c", subcore_axis_name="s")

    @functools.partial(
        pl.kernel, mesh=mesh,
        out_type=[jax.ShapeDtypeStruct((N_ACC, D), jnp.float32)],
        scratch_types=dict(
            acc_sh=pltpu.VMEM_SHARED((N_ACC, D), jnp.float32),
            src_v=pltpu.VMEM((KG, CH), jnp.int32),
            dst_v=pltpu.VMEM((KG, CH), jnp.int32),
            rows_v=pltpu.VMEM((CH, D), jnp.float32),
            sem=pltpu.SemaphoreType.DMA,
        ))
    def seg(y_hbm, src_hbm, dst_hbm, outa_hbm, *, acc_sh, src_v, dst_v,
            rows_v, sem):
        c = lax.axis_index("c")
        s = lax.axis_index("s")
        start = s * RPT0
        ngroups = jnp.where(c == 0, RPT0 // KG, 0)
        zero16 = jnp.zeros((16,), jnp.float32)

        def zrow(i, carry):
            def zcol(k, cc):
                rows_v[i, pl.ds(k * 16, 16)] = zero16
                return cc
            return lax.fori_loop(0, D // 16, zcol, carry)
        lax.fori_loop(0, CH, zrow, 0)

        base = s * PER
        for k in range(PER // CH):
            pltpu.sync_copy(rows_v, acc_sh.at[pl.ds(base + k * CH, CH)])
        rem = PER % CH
        if rem:
            pltpu.sync_copy(rows_v.at[pl.ds(0, rem)],
                            acc_sh.at[pl.ds(base + (PER // CH) * CH, rem)])
        plsc.subcore_barrier()

        def group(g, carry):
            row = start + g * KG
            pltpu.sync_copy(src_hbm.at[pl.ds(row, KG)], src_v)
            pltpu.sync_copy(dst_hbm.at[pl.ds(row, KG)], dst_v)

            def step(j, cc):
                pltpu.async_copy(y_hbm.at[src_v.at[j]], rows_v, sem).wait()
                pltpu.sync_copy(rows_v, acc_sh.at[dst_v.at[j]], add=True)
                return cc
            return lax.fori_loop(0, KG, step, carry)
        lax.fori_loop(0, ngroups, group, 0)
        plsc.subcore_barrier()

        @pl.when(c == 0)
        def _():
            pltpu.sync_copy(acc_sh.at[pl.ds(base, PER)],
                            outa_hbm.at[pl.ds(base, PER)])

    return seg


def _make_cnt():
    mesh = plsc.VectorSubcoreMesh(core_axis_name="c", subcore_axis_name="s")

    @functools.partial(
        pl.kernel, mesh=mesh,
        out_type=[jax.ShapeDtypeStruct((NC, N_ACC, CW), jnp.float32)],
        scratch_types=dict(
            cnt_sh=pltpu.VMEM_SHARED((N_ACC, CW), jnp.float32),
            dst_v=pltpu.VMEM((KG, CH), jnp.int32),
            buf_v=pltpu.VMEM((CH, CW), jnp.float32),
        ))
    def cntk(dst_hbm, outc_hbm, *, cnt_sh, dst_v, buf_v):
        c = lax.axis_index("c")
        s = lax.axis_index("s")
        w = c * NS + s

        def fill(val):
            v16 = jnp.full((16,), val, jnp.float32)

            def frow(i, carry):
                def fcol(k, cc):
                    buf_v[i, pl.ds(k * 16, 16)] = v16
                    return cc
                return lax.fori_loop(0, CW // 16, fcol, carry)
            lax.fori_loop(0, CH, frow, 0)

        fill(0.0)
        base = s * PER
        for k in range(PER // CH):
            pltpu.sync_copy(buf_v, cnt_sh.at[pl.ds(base + k * CH, CH)])
        if PER % CH:
            pltpu.sync_copy(buf_v.at[pl.ds(0, PER % CH)],
                            cnt_sh.at[pl.ds(base + (PER // CH) * CH,
                                            PER % CH)])
        fill(1.0)
        plsc.subcore_barrier()

        def group(g, carry):
            pltpu.sync_copy(dst_hbm.at[pl.ds(w * RPT + g * KG, KG)], dst_v)

            def step(j, cc):
                pltpu.sync_copy(buf_v, cnt_sh.at[dst_v.at[j]], add=True)
                return cc
            return lax.fori_loop(0, KG, step, carry)
        lax.fori_loop(0, RPT // KG, group, 0)
        plsc.subcore_barrier()

        pltpu.sync_copy(cnt_sh.at[pl.ds(base, PER)],
                        outc_hbm.at[c, pl.ds(base, PER)])

    return cntk


_sc_cache = {}


def _seg(*args):
    if "seg" not in _sc_cache:
        _sc_cache["seg"] = _make_seg()
    return _sc_cache["seg"](*args)


def _cnt(*args):
    if "cnt" not in _sc_cache:
        _sc_cache["cnt"] = _make_cnt()
    return _sc_cache["cnt"](*args)


_BR = 1000
_row = pl.BlockSpec((_BR, D), lambda i: (i, 0))
_roww = pl.BlockSpec((_BR, CW), lambda i: (i, 0))
_full = pl.BlockSpec((D, D), lambda i: (0, 0))
_bias = pl.BlockSpec((1, D), lambda i: (0, 0))
_f32 = jnp.float32


def _pre_body(x_ref, wt_ref, y_ref):
    y_ref[...] = jnp.dot(x_ref[...], wt_ref[...],
                         preferred_element_type=_f32)


def _pre(x, wt1l):
    return pl.pallas_call(
        _pre_body,
        grid=(N_NODES // _BR,),
        in_specs=[_row, _full],
        out_specs=_row,
        out_shape=jax.ShapeDtypeStruct((N_NODES, D), _f32),
    )(x, wt1l)


def _mid_body(x_ref, a0_ref, c0_ref, c1_ref, wt1r_ref, b1_ref,
              wt2l_ref, wt2r_ref, b2_ref, y2_ref, r2b_ref):
    cnt = c0_ref[:, 0:1] + c1_ref[:, 0:1]
    inv = 1.0 / jnp.maximum(cnt, 1.0)
    agg = a0_ref[...] * inv
    root = jnp.dot(x_ref[...], wt1r_ref[...], preferred_element_type=_f32)
    h = jnp.maximum(agg + b1_ref[...] + root, 0.0)
    y2_ref[...] = jnp.dot(h, wt2l_ref[...], preferred_element_type=_f32)
    r2b_ref[...] = (jnp.dot(h, wt2r_ref[...], preferred_element_type=_f32)
                    + b2_ref[...])


def _mid(x, a0, c0, c1, wt1r, b1, wt2l, wt2r, b2):
    return pl.pallas_call(
        _mid_body,
        grid=(N_NODES // _BR,),
        in_specs=[_row, _row, _roww, _roww, _full, _bias, _full, _full,
                  _bias],
        out_specs=[_row, _row],
        out_shape=[jax.ShapeDtypeStruct((N_NODES, D), _f32),
                   jax.ShapeDtypeStruct((N_NODES, D), _f32)],
    )(x, a0, c0, c1, wt1r, b1, wt2l, wt2r, b2)


def _fin_body(a0_ref, c0_ref, c1_ref, r2b_ref, out_ref):
    cnt = c0_ref[:, 0:1] + c1_ref[:, 0:1]
    inv = 1.0 / jnp.maximum(cnt, 1.0)
    o = jnp.maximum(a0_ref[...] * inv + r2b_ref[...], 0.0)
    m = jnp.max(o, axis=1, keepdims=True)
    e = jnp.exp(o - m)
    lse = jnp.log(jnp.sum(e, axis=1, keepdims=True))
    out_ref[...] = o - m - lse


def _fin(a0, c0, c1, r2b):
    return pl.pallas_call(
        _fin_body,
        grid=(N_NODES // _BR,),
        in_specs=[_row, _roww, _roww, _row],
        out_specs=_row,
        out_shape=jax.ShapeDtypeStruct((N_NODES, D), _f32),
    )(a0, c0, c1, r2b)


def kernel(x, edge_index, W1_l, b1_l, W1_r, W2_l, b2_l, W2_r):
    src = edge_index[0].astype(jnp.int32)
    dst = edge_index[1].astype(jnp.int32)
    npad = E_PAD - N_EDGES
    src = jnp.concatenate([src, jnp.zeros((npad,), jnp.int32)])
    dst = jnp.concatenate([dst, jnp.full((npad,), N_NODES, jnp.int32)])
    src2 = src.reshape(TOT_ROWS, CH)
    dst2 = dst.reshape(TOT_ROWS, CH)

    y1 = _pre(x, W1_l.T)
    (cnt,) = _cnt(dst2)
    (acc1,) = _seg(y1, src2, dst2)
    c0, c1 = cnt[0, :N_NODES], cnt[1, :N_NODES]
    y2, r2b = _mid(x, acc1[:N_NODES], c0, c1, W1_r.T, b1_l.reshape(1, D),
                   W2_l.T, W2_r.T, b2_l.reshape(1, D))
    (acc2,) = _seg(y2, src2, dst2)
    return _fin(acc2[:N_NODES], c0, c1, r2b)


# speedup vs baseline: 1.4396x; 1.4396x over previous
import functools

import jax
import jax.numpy as jnp
from jax import lax
from jax.experimental import pallas as pl
from jax.experimental.pallas import tpu as pltpu
from jax.experimental.pallas import tpu_sc as plsc

N_NODES = 10000
D = 128
N_EDGES = 320000
NC = 2
NS = 16
NW = NC * NS
CH = 128
RPT = 80
RPT0 = 112
RPT1 = 48
TOT_ROWS = NS * (RPT0 + RPT1)
KG = 8
E_PAD = TOT_ROWS * CH
N_ACC = 10112
PER = N_ACC // NS
CW = 128


def _make_seg():
    mesh = plsc.VectorSubcoreMesh(core_axis_name="c", subcore_axis_name="s")

    @functools.partial(
        pl.kernel, mesh=mesh,
        out_type=[jax.ShapeDtypeStruct((NC, N_ACC, D), jnp.float32)],
        scratch_types=dict(
            acc_sh=pltpu.VMEM_SHARED((N_ACC, D), jnp.float32),
            src_v=pltpu.VMEM((KG, CH), jnp.int32),
            dst_v=pltpu.VMEM((KG, CH), jnp.int32),
            buf_a=pltpu.VMEM((CH, D), jnp.float32),
            buf_b=pltpu.VMEM((CH, D), jnp.float32),
            sem_a=pltpu.SemaphoreType.DMA,
            sem_b=pltpu.SemaphoreType.DMA,
        ))
    def seg(y_hbm, src_hbm, dst_hbm, outa_hbm, *, acc_sh, src_v, dst_v,
            buf_a, buf_b, sem_a, sem_b):
        c = lax.axis_index("c")
        s = lax.axis_index("s")
        start = jnp.where(c == 0, s * RPT0, NS * RPT0 + s * RPT1)
        ngroups = jnp.where(c == 0, RPT0 // KG, RPT1 // KG)
        zero16 = jnp.zeros((16,), jnp.float32)
        bufs = (buf_a, buf_b)
        sems = (sem_a, sem_b)

        def zrow(i, carry):
            def zcol(k, cc):
                buf_a[i, pl.ds(k * 16, 16)] = zero16
                return cc
            return lax.fori_loop(0, D // 16, zcol, carry)
        lax.fori_loop(0, CH, zrow, 0)

        base = s * PER

        def zacc(k, carry):
            pltpu.sync_copy(buf_a, acc_sh.at[pl.ds(base + k * CH, CH)])
            return carry
        lax.fori_loop(0, PER // CH, zacc, 0)
        rem = PER % CH
        if rem:
            pltpu.sync_copy(buf_a.at[pl.ds(0, rem)],
                            acc_sh.at[pl.ds(base + (PER // CH) * CH, rem)])
        plsc.subcore_barrier()

        def group(g, carry):
            row = start + g * KG
            pltpu.sync_copy(src_hbm.at[pl.ds(row, KG)], src_v)
            pltpu.sync_copy(dst_hbm.at[pl.ds(row, KG)], dst_v)
            cps = [None] * KG
            cps[0] = pltpu.async_copy(y_hbm.at[src_v.at[0]], bufs[0],
                                      sems[0])
            for j in range(KG):
                if j + 1 < KG:
                    cps[j + 1] = pltpu.async_copy(
                        y_hbm.at[src_v.at[j + 1]], bufs[(j + 1) % 2],
                        sems[(j + 1) % 2])
                cps[j].wait()
                pltpu.sync_copy(bufs[j % 2], acc_sh.at[dst_v.at[j]],
                                add=True)
            return carry
        lax.fori_loop(0, ngroups, group, 0)
        plsc.subcore_barrier()

        pltpu.sync_copy(acc_sh.at[pl.ds(base, PER)],
                        outa_hbm.at[c, pl.ds(base, PER)])

    return seg


def _make_cnt():
    mesh = plsc.VectorSubcoreMesh(core_axis_name="c", subcore_axis_name="s")

    @functools.partial(
        pl.kernel, mesh=mesh,
        out_type=[jax.ShapeDtypeStruct((NC, N_ACC, CW), jnp.float32)],
        scratch_types=dict(
            cnt_sh=pltpu.VMEM_SHARED((N_ACC, CW), jnp.float32),
            dst_v=pltpu.VMEM((KG, CH), jnp.int32),
            buf_v=pltpu.VMEM((CH, CW), jnp.float32),
        ))
    def cntk(dst_hbm, outc_hbm, *, cnt_sh, dst_v, buf_v):
        c = lax.axis_index("c")
        s = lax.axis_index("s")
        w = c * NS + s

        def fill(val):
            v16 = jnp.full((16,), val, jnp.float32)

            def frow(i, carry):
                def fcol(k, cc):
                    buf_v[i, pl.ds(k * 16, 16)] = v16
                    return cc
                return lax.fori_loop(0, CW // 16, fcol, carry)
            lax.fori_loop(0, CH, frow, 0)

        fill(0.0)
        base = s * PER
        for k in range(PER // CH):
            pltpu.sync_copy(buf_v, cnt_sh.at[pl.ds(base + k * CH, CH)])
        if PER % CH:
            pltpu.sync_copy(buf_v.at[pl.ds(0, PER % CH)],
                            cnt_sh.at[pl.ds(base + (PER // CH) * CH,
                                            PER % CH)])
        fill(1.0)
        plsc.subcore_barrier()

        def group(g, carry):
            pltpu.sync_copy(dst_hbm.at[pl.ds(w * RPT + g * KG, KG)], dst_v)

            def step(j, cc):
                pltpu.sync_copy(buf_v, cnt_sh.at[dst_v.at[j]], add=True)
                return cc
            return lax.fori_loop(0, KG, step, carry)
        lax.fori_loop(0, RPT // KG, group, 0)
        plsc.subcore_barrier()

        pltpu.sync_copy(cnt_sh.at[pl.ds(base, PER)],
                        outc_hbm.at[c, pl.ds(base, PER)])

    return cntk


_sc_cache = {}


def _seg(*args):
    if "seg" not in _sc_cache:
        _sc_cache["seg"] = _make_seg()
    return _sc_cache["seg"](*args)


def _cnt(*args):
    if "cnt" not in _sc_cache:
        _sc_cache["cnt"] = _make_cnt()
    return _sc_cache["cnt"](*args)


_BR = 1000
_row = pl.BlockSpec((_BR, D), lambda i: (i, 0))
_roww = pl.BlockSpec((_BR, CW), lambda i: (i, 0))
_full = pl.BlockSpec((D, D), lambda i: (0, 0))
_bias = pl.BlockSpec((1, D), lambda i: (0, 0))
_f32 = jnp.float32


def _pre_body(x_ref, wt_ref, y_ref):
    y_ref[...] = jnp.dot(x_ref[...], wt_ref[...],
                         preferred_element_type=_f32)


def _pre(x, wt1l):
    return pl.pallas_call(
        _pre_body,
        grid=(N_NODES // _BR,),
        in_specs=[_row, _full],
        out_specs=_row,
        out_shape=jax.ShapeDtypeStruct((N_NODES, D), _f32),
    )(x, wt1l)


def _mid_body(x_ref, a0_ref, a1_ref, c0_ref, c1_ref, wt1r_ref, b1_ref,
              wt2l_ref, wt2r_ref, b2_ref, y2_ref, r2b_ref):
    cnt = c0_ref[:, 0:1] + c1_ref[:, 0:1]
    inv = 1.0 / jnp.maximum(cnt, 1.0)
    agg = (a0_ref[...] + a1_ref[...]) * inv
    root = jnp.dot(x_ref[...], wt1r_ref[...], preferred_element_type=_f32)
    h = jnp.maximum(agg + b1_ref[...] + root, 0.0)
    y2_ref[...] = jnp.dot(h, wt2l_ref[...], preferred_element_type=_f32)
    r2b_ref[...] = (jnp.dot(h, wt2r_ref[...], preferred_element_type=_f32)
                    + b2_ref[...])


def _mid(x, a0, a1, c0, c1, wt1r, b1, wt2l, wt2r, b2):
    return pl.pallas_call(
        _mid_body,
        grid=(N_NODES // _BR,),
        in_specs=[_row, _row, _row, _roww, _roww, _full, _bias, _full,
                  _full, _bias],
        out_specs=[_row, _row],
        out_shape=[jax.ShapeDtypeStruct((N_NODES, D), _f32),
                   jax.ShapeDtypeStruct((N_NODES, D), _f32)],
    )(x, a0, a1, c0, c1, wt1r, b1, wt2l, wt2r, b2)


def _fin_body(a0_ref, a1_ref, c0_ref, c1_ref, r2b_ref, out_ref):
    cnt = c0_ref[:, 0:1] + c1_ref[:, 0:1]
    inv = 1.0 / jnp.maximum(cnt, 1.0)
    o = jnp.maximum((a0_ref[...] + a1_ref[...]) * inv + r2b_ref[...], 0.0)
    m = jnp.max(o, axis=1, keepdims=True)
    e = jnp.exp(o - m)
    lse = jnp.log(jnp.sum(e, axis=1, keepdims=True))
    out_ref[...] = o - m - lse


def _fin(a0, a1, c0, c1, r2b):
    return pl.pallas_call(
        _fin_body,
        grid=(N_NODES // _BR,),
        in_specs=[_row, _row, _roww, _roww, _row],
        out_specs=_row,
        out_shape=jax.ShapeDtypeStruct((N_NODES, D), _f32),
    )(a0, a1, c0, c1, r2b)


def kernel(x, edge_index, W1_l, b1_l, W1_r, W2_l, b2_l, W2_r):
    src = edge_index[0].astype(jnp.int32)
    dst = edge_index[1].astype(jnp.int32)
    npad = E_PAD - N_EDGES
    src = jnp.concatenate([src, jnp.zeros((npad,), jnp.int32)])
    dst = jnp.concatenate([dst, jnp.full((npad,), N_NODES, jnp.int32)])
    src2 = src.reshape(TOT_ROWS, CH)
    dst2 = dst.reshape(TOT_ROWS, CH)

    y1 = _pre(x, W1_l.T)
    (cnt,) = _cnt(dst2)
    (acc1,) = _seg(y1, src2, dst2)
    c0, c1 = cnt[0, :N_NODES], cnt[1, :N_NODES]
    y2, r2b = _mid(x, acc1[0, :N_NODES], acc1[1, :N_NODES], c0, c1,
                   W1_r.T, b1_l.reshape(1, D),
                   W2_l.T, W2_r.T, b2_l.reshape(1, D))
    (acc2,) = _seg(y2, src2, dst2)
    return _fin(acc2[0, :N_NODES], acc2[1, :N_NODES], c0, c1, r2b)
